# baseline (speedup 1.0000x reference)
import functools

import jax
import jax.numpy as jnp
from jax import lax
from jax.experimental import pallas as pl
from jax.experimental.pallas import tpu as pltpu
from jax.experimental.pallas import tpu_sc as plsc

_TC_BN = 1792
_SC_ROWS = 512
_NW = 32
_BIG = 2.0 ** 30


def _tc_body(x1_ref, x2t_ref, d1_ref, i1_ref, d2_ref, i2_ref, *, bn, m):
    ib = pl.program_id(1)

    x1 = x1_ref[0]
    x2t = x2t_ref[0]

    sq1 = jnp.sum(x1 * x1, axis=1, keepdims=True)
    y = -0.5 * x2t
    sq2 = jnp.sum(y * y, axis=0, keepdims=True)

    inner = jnp.dot(x1, x2t, preferred_element_type=jnp.float32)
    d = jnp.maximum((sq1 + sq2) + inner, 0.0)

    rmin = jnp.min(d, axis=1, keepdims=True)
    jcol = lax.broadcasted_iota(jnp.int32, (1, m), 1).astype(jnp.float32)
    rarg = jnp.min(jnp.where(d == rmin, jcol, _BIG), axis=1,
                   keepdims=True).astype(jnp.int32)
    d1_ref[0] = rmin
    i1_ref[0] = rarg

    cmin = jnp.min(d, axis=0, keepdims=True)
    irow = (lax.broadcasted_iota(jnp.int32, (bn, 1), 0)
            + ib * bn).astype(jnp.float32)
    carg = jnp.min(jnp.where(d == cmin, irow, _BIG), axis=0,
                   keepdims=True).astype(jnp.int32)

    @pl.when(ib == 0)
    def _init():
        d2_ref[0] = cmin
        i2_ref[0] = carg

    @pl.when(ib > 0)
    def _update():
        prev_d = d2_ref[0]
        prev_i = i2_ref[0]
        better = cmin < prev_d
        d2_ref[0] = jnp.where(better, cmin, prev_d)
        i2_ref[0] = jnp.where(better, carg, prev_i)


def _tc_call(x1p, x2t, n_tc, m, B):
    bn = _TC_BN
    nb = n_tc // bn
    return pl.pallas_call(
        functools.partial(_tc_body, bn=bn, m=m),
        grid=(B, nb),
        in_specs=[
            pl.BlockSpec((1, bn, 8), lambda b, ib: (b, ib, 0)),
            pl.BlockSpec((1, 8, m), lambda b, ib: (b, 0, 0)),
        ],
        out_specs=(
            pl.BlockSpec((1, bn, 1), lambda b, ib: (b, ib, 0)),
            pl.BlockSpec((1, bn, 1), lambda b, ib: (b, ib, 0)),
            pl.BlockSpec((1, 1, m), lambda b, ib: (b, 0, 0)),
            pl.BlockSpec((1, 1, m), lambda b, ib: (b, 0, 0)),
        ),
        out_shape=(
            jax.ShapeDtypeStruct((B, n_tc, 1), jnp.float32),
            jax.ShapeDtypeStruct((B, n_tc, 1), jnp.int32),
            jax.ShapeDtypeStruct((B, 1, m), jnp.float32),
            jax.ShapeDtypeStruct((B, 1, m), jnp.int32),
        ),
        compiler_params=pltpu.CompilerParams(
            dimension_semantics=("parallel", "arbitrary"),
        ),
    )(x1p, x2t)


def _sc_chamfer(n, m, n_tc, B):
    RW = (B * _SC_ROWS) // _NW
    wpb = _SC_ROWS // RW
    NC = m // 16
    mesh = plsc.VectorSubcoreMesh(core_axis_name="c", subcore_axis_name="s")

    def _round_bf16(v):
        u = lax.bitcast_convert_type(v, jnp.int32)
        r = (u + 0x7FFF + ((u >> 16) & 1)) & jnp.int32(-65536)
        return lax.bitcast_convert_type(r, jnp.float32)

    def body(x1e_h, x2e_h,
             d1_h, i1_h, d2p_h, i2p_h,
             x1es, x2es, x2rs, sq2s, d2s, i2s, d1s, i1s):
        wid = lax.axis_index("c") * 16 + lax.axis_index("s")
        b = wid // wpb
        k = wid % wpb
        row0 = n_tc + k * RW
        start = b * n + row0

        BN = B * n
        for c in range(3):
            pltpu.sync_copy(x1e_h.at[pl.ds(c * BN + start, RW)],
                            x1es.at[pl.ds(c * RW, RW)])
            pltpu.sync_copy(x2e_h.at[pl.ds(c * BN + b * n, m)],
                            x2es.at[pl.ds(c * m, m)])

        inf16 = jnp.full((16,), _BIG, jnp.float32)

        def stage(jc, _):
            sl = pl.ds(jc * 16, 16)
            e0 = x2es[pl.ds(0 * m + jc * 16, 16)]
            e1 = x2es[pl.ds(1 * m + jc * 16, 16)]
            e2 = x2es[pl.ds(2 * m + jc * 16, 16)]
            sq2s[sl] = (e0 * e0 + e1 * e1) + e2 * e2
            x2rs[pl.ds(0 * m + jc * 16, 16)] = _round_bf16(e0)
            x2rs[pl.ds(1 * m + jc * 16, 16)] = _round_bf16(e1)
            x2rs[pl.ds(2 * m + jc * 16, 16)] = _round_bf16(e2)
            d2s[sl] = inf16
            i2s[sl] = inf16
            return 0

        lax.fori_loop(0, NC, stage, 0)

        lane_i = lax.broadcasted_iota(jnp.int32, (16,), 0)
        lanef = lane_i.astype(jnp.float32)

        def rowgroup(g, _):
            gsl = pl.ds(g * 16, 16)
            ev0 = x1es[pl.ds(0 * RW + g * 16, 16)]
            ev1 = x1es[pl.ds(1 * RW + g * 16, 16)]
            ev2 = x1es[pl.ds(2 * RW + g * 16, 16)]
            racc_d = inf16
            racc_i = inf16
            dn = lax.GatherDimensionNumbers(
                offset_dims=(), collapsed_slice_dims=(0,),
                start_index_map=(0,))

            def bcast(v, r):
                idx = jnp.full((16, 1), r, jnp.int32)
                return lax.gather(
                    v, idx, dn, (1,),
                    mode=lax.GatherScatterMode.PROMISE_IN_BOUNDS)

            for r in range(16):
                g0 = bcast(ev0, r)
                g1 = bcast(ev1, r)
                g2 = bcast(ev2, r)
                sq1 = (g0 * g0 + g1 * g1) + g2 * g2
                v0 = _round_bf16(g0)
                v1 = _round_bf16(g1)
                v2 = _round_bf16(g2)
                rowf = jnp.full(
                    (16,), (row0 + g * 16 + r).astype(jnp.float32),
                    jnp.float32)

                def chunk(jc, carry):
                    runm, runi = carry
                    sl = pl.ds(jc * 16, 16)
                    y0 = x2rs[pl.ds(0 * m + jc * 16, 16)]
                    y1 = x2rs[pl.ds(1 * m + jc * 16, 16)]
                    y2 = x2rs[pl.ds(2 * m + jc * 16, 16)]
                    p = (v0 * y0 + v1 * y1) + v2 * y2
                    d = jnp.maximum((sq1 + sq2s[sl]) - 2.0 * p, 0.0)
                    jvec = lanef + (jc * 16).astype(jnp.float32)
                    m1 = d < runm
                    runm = jnp.where(m1, d, runm)
                    runi = jnp.where(m1, jvec, runi)
                    cur = d2s[sl]
                    curi = i2s[sl]
                    m2 = d < cur
                    d2s[sl] = jnp.where(m2, d, cur)
                    i2s[sl] = jnp.where(m2, rowf, curi)
                    return runm, runi

                runm, runi = lax.fori_loop(0, NC, chunk, (inf16, inf16))
                for st in (8, 4, 2, 1):
                    perm = (lane_i ^ st)[:, None]
                    om = lax.gather(
                        runm, perm, dn, (1,),
                        mode=lax.GatherScatterMode.PROMISE_IN_BOUNDS)
                    oi = lax.gather(
                        runi, perm, dn, (1,),
                        mode=lax.GatherScatterMode.PROMISE_IN_BOUNDS)
                    bet = (om < runm) | ((om == runm) & (oi < runi))
                    runm = jnp.where(bet, om, runm)
                    runi = jnp.where(bet, oi, runi)
                msk = lane_i == r
                racc_d = jnp.where(msk, runm, racc_d)
                racc_i = jnp.where(msk, runi, racc_i)
            d1s[gsl] = racc_d
            i1s[gsl] = racc_i
            return 0

        lax.fori_loop(0, RW // 16, rowgroup, 0)

        pltpu.sync_copy(d1s, d1_h.at[pl.ds(wid * RW, RW)])
        pltpu.sync_copy(i1s, i1_h.at[pl.ds(wid * RW, RW)])
        pltpu.sync_copy(d2s, d2p_h.at[pl.ds(wid * m, m)])
        pltpu.sync_copy(i2s, i2p_h.at[pl.ds(wid * m, m)])

    return pl.kernel(
        body,
        mesh=mesh,
        out_type=(
            jax.ShapeDtypeStruct((B * _SC_ROWS,), jnp.float32),
            jax.ShapeDtypeStruct((B * _SC_ROWS,), jnp.float32),
            jax.ShapeDtypeStruct((_NW * m,), jnp.float32),
            jax.ShapeDtypeStruct((_NW * m,), jnp.float32),
        ),
        scratch_types=[
            pltpu.VMEM((3 * RW,), jnp.float32),
            pltpu.VMEM((3 * m,), jnp.float32),
            pltpu.VMEM((3 * m,), jnp.float32),
            pltpu.VMEM((m,), jnp.float32),
            pltpu.VMEM((m,), jnp.float32),
            pltpu.VMEM((m,), jnp.float32),
            pltpu.VMEM((RW,), jnp.float32),
            pltpu.VMEM((RW,), jnp.float32),
        ],
    )


def _merge_body(d2t_ref, i2t_ref, d2p_ref, i2p_ref, do_ref, io_ref, *, wpb):
    curd = d2t_ref[0]
    curi = i2t_ref[0].astype(jnp.float32)
    dp = d2p_ref[0]
    ip = i2p_ref[0]
    for k in range(wpb):
        dk = dp[k:k + 1, :]
        ik = ip[k:k + 1, :]
        better = (dk < curd) | ((dk == curd) & (ik < curi))
        curd = jnp.where(better, dk, curd)
        curi = jnp.where(better, ik, curi)
    do_ref[0] = curd
    io_ref[0] = curi.astype(jnp.int32)


def _merge_call(d2t, i2t, d2p, i2p, m, B, wpb):
    return pl.pallas_call(
        functools.partial(_merge_body, wpb=wpb),
        grid=(B,),
        in_specs=[
            pl.BlockSpec((1, 1, m), lambda b: (b, 0, 0)),
            pl.BlockSpec((1, 1, m), lambda b: (b, 0, 0)),
            pl.BlockSpec((1, wpb, m), lambda b: (b, 0, 0)),
            pl.BlockSpec((1, wpb, m), lambda b: (b, 0, 0)),
        ],
        out_specs=(
            pl.BlockSpec((1, 1, m), lambda b: (b, 0, 0)),
            pl.BlockSpec((1, 1, m), lambda b: (b, 0, 0)),
        ),
        out_shape=(
            jax.ShapeDtypeStruct((B, 1, m), jnp.float32),
            jax.ShapeDtypeStruct((B, 1, m), jnp.int32),
        ),
    )(d2t, i2t, d2p, i2p)


@jax.jit
def kernel(xyz1, xyz2):
    B, n, _ = xyz1.shape
    m = xyz2.shape[1]
    n_tc = n - _SC_ROWS
    wpb = _SC_ROWS // ((B * _SC_ROWS) // _NW)

    x1p = jnp.pad(xyz1[:, :n_tc], ((0, 0), (0, 0), (0, 5)))
    x2t = jnp.pad(jnp.transpose(-2.0 * xyz2, (0, 2, 1)),
                  ((0, 0), (0, 5), (0, 0)))

    x1e = jnp.transpose(xyz1.reshape(B * n, 3), (1, 0)).reshape(-1)
    x2e = jnp.transpose(xyz2.reshape(B * n, 3), (1, 0)).reshape(-1)

    d1t, i1t, d2t, i2t = _tc_call(x1p, x2t, n_tc, m, B)
    d1s, i1s, d2p, i2p = _sc_chamfer(n, m, n_tc, B)(x1e, x2e)

    d2, i2 = _merge_call(d2t, i2t, d2p.reshape(B, wpb, m),
                         i2p.reshape(B, wpb, m), m, B, wpb)

    dist1 = jnp.concatenate([d1t[:, :, 0], d1s.reshape(B, _SC_ROWS)], axis=1)
    idx1 = jnp.concatenate(
        [i1t[:, :, 0], i1s.reshape(B, _SC_ROWS).astype(jnp.int32)], axis=1)
    return (dist1, d2[:, 0, :], idx1, i2[:, 0, :])


# speedup vs baseline: 1.6257x; 1.6257x over previous
import functools

import jax
import jax.numpy as jnp
from jax.experimental import pallas as pl
from jax.experimental.pallas import tpu as pltpu

_BN = 2048


def _chamfer_body(x1_ref, x2t_ref, d1_ref, i1_ref, d2_ref, i2_ref, *, bn, m):
    ib = pl.program_id(1)

    x1 = x1_ref[0]
    x2t = x2t_ref[0]

    sq1 = jnp.sum(x1 * x1, axis=1, keepdims=True)
    y = -0.5 * x2t
    sq2 = jnp.sum(y * y, axis=0, keepdims=True)

    inner = jnp.dot(x1, x2t, preferred_element_type=jnp.float32)
    d = jnp.maximum((sq1 + sq2) + inner, 0.0)

    big = jnp.float32(2.0 ** 30)

    rmin = jnp.min(d, axis=1, keepdims=True)
    jcol = jax.lax.broadcasted_iota(
        jnp.int32, (1, m), 1).astype(jnp.float32)
    rarg = jnp.min(jnp.where(d == rmin, jcol, big), axis=1,
                   keepdims=True).astype(jnp.int32)
    d1_ref[0] = rmin
    i1_ref[0] = rarg

    cmin = jnp.min(d, axis=0, keepdims=True)
    irow = (jax.lax.broadcasted_iota(jnp.int32, (bn, 1), 0)
            + ib * bn).astype(jnp.float32)
    carg = jnp.min(jnp.where(d == cmin, irow, big), axis=0,
                   keepdims=True).astype(jnp.int32)

    @pl.when(ib == 0)
    def _init():
        d2_ref[0] = cmin
        i2_ref[0] = carg

    @pl.when(ib > 0)
    def _update():
        prev_d = d2_ref[0]
        prev_i = i2_ref[0]
        better = cmin < prev_d
        d2_ref[0] = jnp.where(better, cmin, prev_d)
        i2_ref[0] = jnp.where(better, carg, prev_i)


@jax.jit
def kernel(xyz1, xyz2):
    B, n, _ = xyz1.shape
    m = xyz2.shape[1]
    bn = _BN
    nb = n // bn

    x1p = jnp.pad(xyz1, ((0, 0), (0, 0), (0, 5)))
    x2t = jnp.pad(jnp.transpose(-2.0 * xyz2, (0, 2, 1)),
                  ((0, 0), (0, 5), (0, 0)))

    grid = (B, nb)
    out_shapes = (
        jax.ShapeDtypeStruct((B, n, 1), jnp.float32),
        jax.ShapeDtypeStruct((B, n, 1), jnp.int32),
        jax.ShapeDtypeStruct((B, 1, m), jnp.float32),
        jax.ShapeDtypeStruct((B, 1, m), jnp.int32),
    )
    in_specs = [
        pl.BlockSpec((1, bn, 8), lambda b, ib: (b, ib, 0)),
        pl.BlockSpec((1, 8, m), lambda b, ib: (b, 0, 0)),
    ]
    out_specs = (
        pl.BlockSpec((1, bn, 1), lambda b, ib: (b, ib, 0)),
        pl.BlockSpec((1, bn, 1), lambda b, ib: (b, ib, 0)),
        pl.BlockSpec((1, 1, m), lambda b, ib: (b, 0, 0)),
        pl.BlockSpec((1, 1, m), lambda b, ib: (b, 0, 0)),
    )

    d1, i1, d2, i2 = pl.pallas_call(
        functools.partial(_chamfer_body, bn=bn, m=m),
        grid=grid,
        in_specs=in_specs,
        out_specs=out_specs,
        out_shape=out_shapes,
        compiler_params=pltpu.CompilerParams(
            dimension_semantics=("parallel", "arbitrary"),
        ),
    )(x1p, x2t)

    return (d1[:, :, 0], d2[:, 0, :], i1[:, :, 0], i2[:, 0, :])
